# baseline (speedup 1.0000x reference)
import functools

import jax
import jax.numpy as jnp
from jax import lax
from jax.experimental import pallas as pl
from jax.experimental.pallas import tpu as pltpu
from jax.experimental.pallas import tpu_sc as plsc

BN_EPS = 1e-5
NUM_GRAPHS = 64
LANE = 128
NB = 256
EB = 512
SC_TILE = 56


def _f32(*args):
    return [jnp.asarray(a, jnp.float32) for a in args]


def _sc_aggregate(idx2, h_chunks, ee_chunks):
    nc = len(h_chunks)
    Np = h_chunks[0].shape[0]
    Ep = ee_chunks[0].shape[0]
    ee_chunks = [lax.bitcast_convert_type(e.reshape(Ep, LANE // 2, 2),
                                          jnp.int32) for e in ee_chunks]
    T = SC_TILE
    cpc = nc // 2
    NT = Ep // T // 16
    RPT = Np // 16
    ZR = 32

    mesh = plsc.VectorSubcoreMesh(core_axis_name="c", subcore_axis_name="s")

    def body(*refs):
        idx_r = refs[0]
        hcs = refs[1:1 + nc]
        ecs = refs[1 + nc:1 + 2 * nc]
        outs = refs[1 + 2 * nc:1 + 3 * nc]
        (idxb_r, hrow_r, erow_r, outb_r, zbuf_r, acc_r,
         sg0, sg1, se0, se1, ss0, ss1, ix0, ix1, ix2, ix3) = refs[1 + 3 * nc:]
        sg = (sg0, sg1)
        se = (se0, se1)
        ss = (ss0, ss1)
        ix = (ix0, ix1, ix2, ix3)
        c = lax.axis_index("c")
        s = lax.axis_index("s")

        def zrow(r, u):
            for g in range(LANE // 16):
                zbuf_r[r, pl.ds(g * 16, 16)] = jnp.zeros((16,), jnp.float32)
            return u
        lax.fori_loop(0, ZR, zrow, 0)

        def run_chunk(h_hbm, e_hbm, out_hbm):
            for j in range(RPT // ZR):
                pltpu.sync_copy(zbuf_r, acc_r.at[pl.ds(s * RPT + j * ZR, ZR)])
            plsc.subcore_barrier()

            def load_idx(t, sl):
                pltpu.async_copy(idx_r.at[s * NT + t], idxb_r.at[sl], ix[sl])

            def issue(t, isl, bsl):
                pltpu.async_copy(h_hbm.at[idxb_r.at[isl, 0]],
                                 hrow_r.at[bsl], sg[bsl])
                pltpu.async_copy(e_hbm.at[pl.ds((s * NT + t) * T, T)],
                                 erow_r.at[bsl], se[bsl])

            load_idx(0, 0)
            load_idx(1, 1)
            pltpu.make_async_copy(idx_r.at[s * NT], idxb_r.at[0], ix[0]).wait()
            issue(0, 0, 0)

            def quad(g4, u):
                for b in range(4):
                    t = 4 * g4 + b
                    bs = b % 2
                    i1 = (b + 1) % 4
                    i2 = (b + 2) % 4

                    @pl.when(t + 1 < NT)
                    def _():
                        pltpu.make_async_copy(idx_r.at[s * NT + t + 1],
                                              idxb_r.at[i1], ix[i1]).wait()
                        issue(t + 1, i1, 1 - bs)

                    pltpu.make_async_copy(h_hbm.at[idxb_r.at[b, 0]],
                                          hrow_r.at[bs], sg[bs]).wait()
                    pltpu.make_async_copy(
                        e_hbm.at[pl.ds((s * NT + t) * T, T)],
                        erow_r.at[bs], se[bs]).wait()

                    @pl.when(t >= 2)
                    def _():
                        pltpu.make_async_copy(outb_r.at[bs],
                                              acc_r.at[idxb_r.at[b, 1]],
                                              ss[bs]).wait()

                    @pl.when(t + 2 < NT)
                    def _():
                        load_idx(t + 2, i2)

                    def rowfn(r, v):
                        for g in range(LANE // 32):
                            u = erow_r[bs, r, pl.ds(g * 16, 16)]
                            elo = lax.bitcast_convert_type(
                                lax.shift_left(u, jnp.int32(16)), jnp.float32)
                            ehi = lax.bitcast_convert_type(
                                lax.bitwise_and(u, jnp.int32(-65536)),
                                jnp.float32)
                            sl0 = pl.ds(g * 32, 16)
                            sl1 = pl.ds(g * 32 + 16, 16)
                            outb_r[bs, r, sl0] = jnp.maximum(
                                hrow_r[bs, r, sl0] + elo, 0.0)
                            outb_r[bs, r, sl1] = jnp.maximum(
                                hrow_r[bs, r, sl1] + ehi, 0.0)
                        return v
                    lax.fori_loop(0, T, rowfn, 0)
                    pltpu.async_copy(outb_r.at[bs], acc_r.at[idxb_r.at[b, 1]],
                                     ss[bs], add=True)
                return u
            lax.fori_loop(0, NT // 4, quad, 0)
            for b in range(2):
                t = NT - 2 + b
                pltpu.make_async_copy(outb_r.at[b], acc_r.at[idxb_r.at[(t % 4), 1]],
                                      ss[b]).wait()
            plsc.subcore_barrier()
            pltpu.sync_copy(acc_r.at[pl.ds(s * RPT, RPT)],
                            out_hbm.at[pl.ds(s * RPT, RPT)])
            plsc.subcore_barrier()

        for core in range(2):
            @pl.when(c == core)
            def _():
                for k in range(cpc):
                    ch = core * cpc + k
                    run_chunk(hcs[ch], ecs[ch], outs[ch])

    fn = pl.kernel(
        body,
        out_type=tuple(jax.ShapeDtypeStruct((Np, LANE), jnp.float32)
                       for _ in range(nc)),
        mesh=mesh,
        scratch_types=[
            pltpu.VMEM((4, 2, T), jnp.int32),
            pltpu.VMEM((2, T, LANE), jnp.float32),
            pltpu.VMEM((2, T, LANE // 2), jnp.int32),
            pltpu.VMEM((2, T, LANE), jnp.float32),
            pltpu.VMEM((ZR, LANE), jnp.float32),
            pltpu.VMEM_SHARED((Np, LANE), jnp.float32),
        ] + [pltpu.SemaphoreType.DMA] * 10,
    )
    return list(fn(idx2, *h_chunks, *ee_chunks))


def _edge_call(ea_p, W1, b1, W2, b2, elW0, elb0):
    Ep, DE = ea_p.shape
    EMH = W1.shape[1]
    d0 = elW0.shape[1]
    n0 = d0 // LANE
    grid = (Ep // EB,)

    def bodyfn(ea_r, W1_r, b1_r, W2_r, b2_r, eW_r, eb_r, e_r, *outs):
        t = jnp.maximum(
            jnp.dot(ea_r[...], W1_r[...], preferred_element_type=jnp.float32)
            + b1_r[...], 0.0)
        e = jnp.dot(t, W2_r[...], preferred_element_type=jnp.float32) + b2_r[...]
        e_r[...] = e
        ee = jnp.dot(e, eW_r[...], preferred_element_type=jnp.float32) + eb_r[...]
        ee = ee.astype(jnp.bfloat16)
        for k, o in enumerate(outs):
            o[...] = ee[:, k * LANE:(k + 1) * LANE]

    return pl.pallas_call(
        bodyfn,
        grid=grid,
        in_specs=[
            pl.BlockSpec((EB, DE), lambda i: (i, 0)),
            pl.BlockSpec((DE, EMH), lambda i: (0, 0)),
            pl.BlockSpec((1, EMH), lambda i: (0, 0)),
            pl.BlockSpec((EMH, DE), lambda i: (0, 0)),
            pl.BlockSpec((1, DE), lambda i: (0, 0)),
            pl.BlockSpec((DE, d0), lambda i: (0, 0)),
            pl.BlockSpec((1, d0), lambda i: (0, 0)),
        ],
        out_specs=[pl.BlockSpec((EB, DE), lambda i: (i, 0))]
        + [pl.BlockSpec((EB, LANE), lambda i: (i, 0))] * n0,
        out_shape=[jax.ShapeDtypeStruct((Ep, DE), jnp.float32)]
        + [jax.ShapeDtypeStruct((Ep, LANE), jnp.bfloat16)] * n0,
    )(ea_p, W1, b1, W2, b2, elW0, elb0)


def _ee_call(e_p, elW, elb):
    Ep, DE = e_p.shape
    d = elW.shape[1]
    n = d // LANE
    grid = (Ep // EB,)

    def bodyfn(e_r, W_r, b_r, *outs):
        ee = jnp.dot(e_r[...], W_r[...], preferred_element_type=jnp.float32) + b_r[...]
        ee = ee.astype(jnp.bfloat16)
        for k, o in enumerate(outs):
            o[...] = ee[:, k * LANE:(k + 1) * LANE]

    return pl.pallas_call(
        bodyfn,
        grid=grid,
        in_specs=[
            pl.BlockSpec((EB, DE), lambda i: (i, 0)),
            pl.BlockSpec((DE, d), lambda i: (0, 0)),
            pl.BlockSpec((1, d), lambda i: (0, 0)),
        ],
        out_specs=[pl.BlockSpec((EB, LANE), lambda i: (i, 0))] * n,
        out_shape=[jax.ShapeDtypeStruct((Ep, LANE), jnp.bfloat16)] * n,
    )(e_p, elW, elb)


def _node_call(h_chunks, a_chunks, W1, b1, W2, b2, W3p, b3p, eps2d, residual):
    nh = len(h_chunks)
    Np = h_chunks[0].shape[0]
    din = nh * LANE
    Hd = W1.shape[1]
    nout = Hd // LANE
    grid = (Np // NB,)

    def bodyfn(*refs):
        hrs = refs[:nh]
        ars = refs[nh:2 * nh]
        W1_r, b1_r, W2_r, b2_r, W3_r, b3_r, eps_r = refs[2 * nh:2 * nh + 7]
        outs = refs[2 * nh + 7:]
        h = jnp.concatenate([r[...] for r in hrs], axis=1)
        ag = jnp.concatenate([r[...] for r in ars], axis=1)
        z = (1.0 + eps_r[0, 0]) * h + ag
        z = jnp.maximum(
            jnp.dot(z, W1_r[...], preferred_element_type=jnp.float32) + b1_r[...], 0.0)
        z = jnp.maximum(
            jnp.dot(z, W2_r[...], preferred_element_type=jnp.float32) + b2_r[...], 0.0)
        z = jnp.dot(z, W3_r[...], preferred_element_type=jnp.float32) + b3_r[...]
        z = jnp.maximum(z, 0.0)
        if residual:
            z = h + z
        for k, o in enumerate(outs):
            o[...] = z[:, k * LANE:(k + 1) * LANE]

    return list(pl.pallas_call(
        bodyfn,
        grid=grid,
        in_specs=[pl.BlockSpec((NB, LANE), lambda i: (i, 0))] * (2 * nh)
        + [
            pl.BlockSpec((din, Hd), lambda i: (0, 0)),
            pl.BlockSpec((1, Hd), lambda i: (0, 0)),
            pl.BlockSpec((Hd, Hd), lambda i: (0, 0)),
            pl.BlockSpec((1, Hd), lambda i: (0, 0)),
            pl.BlockSpec((Hd, Hd), lambda i: (0, 0)),
            pl.BlockSpec((1, Hd), lambda i: (0, 0)),
            pl.BlockSpec((1, 1), lambda i: (0, 0)),
        ],
        out_specs=[pl.BlockSpec((NB, LANE), lambda i: (i, 0))] * nout,
        out_shape=[jax.ShapeDtypeStruct((Np, LANE), jnp.float32)] * nout,
    )(*h_chunks, *a_chunks, W1, b1, W2, b2, W3p, b3p, eps2d))


def _pool_call(h3_chunks, h2_chunks, batch2d, aW1, ab1, aW2, ab2, jW, jb):
    nh = len(h3_chunks)
    Np = h3_chunks[0].shape[0]
    Hd = nh * LANE
    Ah = aW1.shape[1]
    Jd = jW.shape[1]
    grid = (Np // NB,)

    def bodyfn(*refs):
        h3rs = refs[:nh]
        h2rs = refs[nh:2 * nh]
        b_r, aW1_r, ab1_r, aW2_r, ab2_r, jW_r, jb_r = refs[2 * nh:2 * nh + 7]
        pooled_r, jpool_r, counts_r, z_r = refs[2 * nh + 7:]
        i = pl.program_id(0)

        @pl.when(i == 0)
        def _():
            pooled_r[...] = jnp.zeros_like(pooled_r)
            jpool_r[...] = jnp.zeros_like(jpool_r)
            counts_r[...] = jnp.zeros_like(counts_r)
            z_r[...] = jnp.zeros_like(z_r)

        h3 = jnp.concatenate([r[...] for r in h3rs], axis=1)
        h2 = jnp.concatenate([r[...] for r in h2rs], axis=1)
        b = b_r[...]
        t = jnp.tanh(
            jnp.dot(h3, aW1_r[...], preferred_element_type=jnp.float32) + ab1_r[...])
        a = jnp.dot(t, aW2_r[...], preferred_element_type=jnp.float32) + ab2_r[...]
        ea = jnp.exp(a)
        valid = b < NUM_GRAPHS
        eam = jnp.where(valid, ea, 0.0)
        onehot = (b == lax.broadcasted_iota(jnp.int32, (NB, NUM_GRAPHS), 1)
                  ).astype(jnp.float32)
        hw = h3 * eam
        pooled_r[...] += lax.dot_general(
            onehot, hw, (((0,), (0,)), ((), ())),
            preferred_element_type=jnp.float32)
        jout = jnp.dot(h2, jW_r[...], preferred_element_type=jnp.float32) + jb_r[...]
        jpool_r[...] += lax.dot_general(
            onehot, jout, (((0,), (0,)), ((), ())),
            preferred_element_type=jnp.float32)
        counts_r[...] += jnp.sum(onehot, axis=0).reshape(NUM_GRAPHS, 1)
        z_r[...] += jnp.sum(eam).reshape(1, 1)

    cmap = lambda i: (0, 0)
    return pl.pallas_call(
        bodyfn,
        grid=grid,
        in_specs=[pl.BlockSpec((NB, LANE), lambda i: (i, 0))] * (2 * nh)
        + [
            pl.BlockSpec((NB, 1), lambda i: (i, 0)),
            pl.BlockSpec((Hd, Ah), cmap),
            pl.BlockSpec((1, Ah), cmap),
            pl.BlockSpec((Ah, 1), cmap),
            pl.BlockSpec((1, 1), cmap),
            pl.BlockSpec((Hd, Jd), cmap),
            pl.BlockSpec((1, Jd), cmap),
        ],
        out_specs=[
            pl.BlockSpec((NUM_GRAPHS, Hd), cmap),
            pl.BlockSpec((NUM_GRAPHS, Jd), cmap),
            pl.BlockSpec((NUM_GRAPHS, 1), cmap),
            pl.BlockSpec((1, 1), cmap),
        ],
        out_shape=[
            jax.ShapeDtypeStruct((NUM_GRAPHS, Hd), jnp.float32),
            jax.ShapeDtypeStruct((NUM_GRAPHS, Jd), jnp.float32),
            jax.ShapeDtypeStruct((NUM_GRAPHS, 1), jnp.float32),
            jax.ShapeDtypeStruct((1, 1), jnp.float32),
        ],
    )(*h3_chunks, *h2_chunks, batch2d, aW1, ab1, aW2, ab2, jW, jb)


def _head_call(pooled_u, jpool_u, counts, zsum, jcWa, jcWb, jcb,
               mW1, mb1, s1, t1, mW2, mb2, s2, t2, mW3, mb3, s3, t3,
               mW4, mb4, temp):
    OUTD = mW4.shape[1]

    def bodyfn(pool_r, jp_r, cnt_r, z_r, jcWa_r, jcWb_r, jcb_r,
               W1_r, b1_r, s1_r, t1_r, W2_r, b2_r, s2_r, t2_r,
               W3_r, b3_r, s3_r, t3_r, W4_r, b4_r, temp_r, out_r):
        pooled = pool_r[...] / z_r[0, 0]
        cnt = jnp.maximum(cnt_r[...], 1.0)
        jp = jp_r[...] / cnt
        xp = (jnp.dot(pooled, jcWa_r[...], preferred_element_type=jnp.float32)
              + jnp.dot(jp, jcWb_r[...], preferred_element_type=jnp.float32)
              + jcb_r[...])

        def bn_block(y, W_r, b_r, s_r, t_r):
            y = jnp.maximum(
                jnp.dot(y, W_r[...], preferred_element_type=jnp.float32) + b_r[...],
                0.0)
            return y * s_r[...] + t_r[...]

        y = bn_block(xp, W1_r, b1_r, s1_r, t1_r)
        y = bn_block(y, W2_r, b2_r, s2_r, t2_r)
        y = bn_block(y, W3_r, b3_r, s3_r, t3_r)
        logits = jnp.dot(y, W4_r[...], preferred_element_type=jnp.float32) + b4_r[...]
        out_r[...] = logits / temp_r[0, 0]

    return pl.pallas_call(
        bodyfn,
        out_shape=jax.ShapeDtypeStruct((NUM_GRAPHS, OUTD), jnp.float32),
    )(pooled_u, jpool_u, counts, zsum, jcWa, jcWb, jcb,
      mW1, mb1, s1, t1, mW2, mb2, s2, t2, mW3, mb3, s3, t3, mW4, mb4, temp)


def _chunks(arr):
    return [arr[:, k * LANE:(k + 1) * LANE] for k in range(arr.shape[1] // LANE)]


def _r2(v):
    return jnp.asarray(v, jnp.float32).reshape(1, -1)


def kernel(x, edge_index, edge_attr, batch, params):
    p = params
    N, D = x.shape
    E = edge_index.shape[1]
    Hd = p['g0_W1'].shape[1]

    Np = ((N + 2047) // 2048 + (1 if N % 2048 == 0 else 0)) * 2048
    epq = 16 * SC_TILE * 4
    while epq % EB:
        epq *= 2
    Ep = ((E + epq - 1) // epq) * epq

    src_p = jnp.concatenate(
        [edge_index[0], jnp.zeros((Ep - E,), jnp.int32)]).reshape(-1, SC_TILE)
    dst_p = jnp.concatenate(
        [edge_index[1], jnp.full((Ep - E,), N, jnp.int32)]).reshape(-1, SC_TILE)
    idx2 = jnp.stack([src_p, dst_p], axis=1)
    ea_p = jnp.concatenate(
        [edge_attr, jnp.zeros((Ep - E, edge_attr.shape[1]), jnp.float32)])
    x_p = jnp.concatenate([x, jnp.zeros((Np - N, D), jnp.float32)])
    batch_p = jnp.concatenate(
        [batch, jnp.full((Np - N,), NUM_GRAPHS, jnp.int32)]).reshape(Np, 1)

    def _ilv(W, b):
        d = W.shape[1]
        idx = jnp.arange(d)
        r = idx % 32
        perm = (idx // 32) * 32 + jnp.where(r % 2 == 0, r // 2, 16 + r // 2)
        return W[:, perm], _r2(b[perm])

    eW0, eb0 = _ilv(p['g0_el_W'], p['g0_el_b'])
    eW1, eb1 = _ilv(p['g1_el_W'], p['g1_el_b'])
    eW2, eb2 = _ilv(p['g2_el_W'], p['g2_el_b'])
    e_p, *ee0c = _edge_call(ea_p, p['emlp_W1'], _r2(p['emlp_b1']),
                            p['emlp_W2'], _r2(p['emlp_b2']), eW0, eb0)
    ee1c = list(_ee_call(e_p, eW1, eb1))
    ee2c = list(_ee_call(e_p, eW2, eb2))

    def fold_bn(i):
        scale = (p['g%d_bn_g' % i] / jnp.sqrt(1.0 + BN_EPS))
        W3p = p['g%d_W3' % i] * scale[None, :]
        b3p = p['g%d_b3' % i] * scale + p['g%d_bn_b' % i]
        return W3p, _r2(b3p)

    hc = _chunks(x_p)
    for i, eec in enumerate((ee0c, ee1c, ee2c)):
        aggr = _sc_aggregate(idx2, hc, eec)
        W3p, b3p = fold_bn(i)
        out = _node_call(hc, aggr,
                         p['g%d_W1' % i], _r2(p['g%d_b1' % i]),
                         p['g%d_W2' % i], _r2(p['g%d_b2' % i]),
                         W3p, b3p,
                         p['g%d_eps' % i].reshape(1, 1),
                         residual=(i > 0))
        if i == 1:
            h2c = out
        hc = out
    h3c = hc

    pooled_u, jpool_u, counts, zsum = _pool_call(
        h3c, h2c, batch_p,
        p['attn_W1'], _r2(p['attn_b1']), p['attn_W2'], _r2(p['attn_b2']),
        p['jump_W'], _r2(p['jump_b']))

    jc_Wa = p['jc_W'][:Hd]
    jc_Wb = p['jc_W'][Hd:]

    def mbn(i):
        s = (p['m_bn%d_g' % i] / jnp.sqrt(1.0 + BN_EPS))
        return _r2(s), _r2(p['m_bn%d_b' % i])

    s1, t1 = mbn(1)
    s2, t2 = mbn(2)
    s3, t3 = mbn(3)
    logits = _head_call(
        pooled_u, jpool_u, counts, zsum, jc_Wa, jc_Wb, _r2(p['jc_b']),
        p['m_W1'], _r2(p['m_b1']), s1, t1,
        p['m_W2'], _r2(p['m_b2']), s2, t2,
        p['m_W3'], _r2(p['m_b3']), s3, t3,
        p['m_W4'], _r2(p['m_b4']), p['temperature'].reshape(1, 1))
    return logits


# speedup vs baseline: 4.4524x; 4.4524x over previous
import functools

import jax
import jax.numpy as jnp
from jax import lax
from jax.experimental import pallas as pl
from jax.experimental.pallas import tpu as pltpu
from jax.experimental.pallas import tpu_sc as plsc

BN_EPS = 1e-5
NUM_GRAPHS = 64
LANE = 128
NB = 256
EB = 512
SC_TILE = 56


def _f32(*args):
    return [jnp.asarray(a, jnp.float32) for a in args]


def _sc_aggregate(idx2, h_chunks, ee_chunks):
    nc = len(h_chunks)
    Np = h_chunks[0].shape[0]
    Ep = ee_chunks[0].shape[0]
    T = SC_TILE
    cpc = nc // 2
    NT = Ep // T // 16
    RPT = Np // 16
    ZR = 32

    mesh = plsc.VectorSubcoreMesh(core_axis_name="c", subcore_axis_name="s")

    def body(*refs):
        idx_r = refs[0]
        hcs = refs[1:1 + nc]
        ecs = refs[1 + nc:1 + 2 * nc]
        outs = refs[1 + 2 * nc:1 + 3 * nc]
        (idxb_r, hrow_r, erow_r, outb_r, zbuf_r, acc_r,
         sg0, sg1, se0, se1, ss0, ss1, ix0, ix1, ix2, ix3) = refs[1 + 3 * nc:]
        sg = (sg0, sg1)
        se = (se0, se1)
        ss = (ss0, ss1)
        ix = (ix0, ix1, ix2, ix3)
        c = lax.axis_index("c")
        s = lax.axis_index("s")

        def zrow(r, u):
            for g in range(LANE // 16):
                zbuf_r[r, pl.ds(g * 16, 16)] = jnp.zeros((16,), jnp.float32)
            return u
        lax.fori_loop(0, ZR, zrow, 0)

        def run_chunk(h_hbm, e_hbm, out_hbm):
            for j in range(RPT // ZR):
                pltpu.sync_copy(zbuf_r, acc_r.at[pl.ds(s * RPT + j * ZR, ZR)])
            plsc.subcore_barrier()

            def load_idx(t, sl):
                pltpu.async_copy(idx_r.at[s * NT + t], idxb_r.at[sl], ix[sl])

            def issue(t, isl, bsl):
                pltpu.async_copy(h_hbm.at[idxb_r.at[isl, 0]],
                                 hrow_r.at[bsl], sg[bsl])
                pltpu.async_copy(e_hbm.at[pl.ds((s * NT + t) * T, T)],
                                 erow_r.at[bsl], se[bsl])

            load_idx(0, 0)
            load_idx(1, 1)
            pltpu.make_async_copy(idx_r.at[s * NT], idxb_r.at[0], ix[0]).wait()
            issue(0, 0, 0)

            def quad(g4, u):
                for b in range(4):
                    t = 4 * g4 + b
                    bs = b % 2
                    i1 = (b + 1) % 4
                    i2 = (b + 2) % 4

                    @pl.when(t + 1 < NT)
                    def _():
                        pltpu.make_async_copy(idx_r.at[s * NT + t + 1],
                                              idxb_r.at[i1], ix[i1]).wait()
                        issue(t + 1, i1, 1 - bs)

                    pltpu.make_async_copy(h_hbm.at[idxb_r.at[b, 0]],
                                          hrow_r.at[bs], sg[bs]).wait()
                    pltpu.make_async_copy(
                        e_hbm.at[pl.ds((s * NT + t) * T, T)],
                        erow_r.at[bs], se[bs]).wait()

                    @pl.when(t >= 2)
                    def _():
                        pltpu.make_async_copy(outb_r.at[bs],
                                              acc_r.at[idxb_r.at[b, 1]],
                                              ss[bs]).wait()

                    @pl.when(t + 2 < NT)
                    def _():
                        load_idx(t + 2, i2)

                    def rowfn(r, v):
                        for g in range(LANE // 16):
                            sl = pl.ds(g * 16, 16)
                            outb_r[bs, r, sl] = jnp.maximum(
                                hrow_r[bs, r, sl] + erow_r[bs, r, sl], 0.0)
                        return v
                    lax.fori_loop(0, T, rowfn, 0)
                    pltpu.async_copy(outb_r.at[bs], acc_r.at[idxb_r.at[b, 1]],
                                     ss[bs], add=True)
                return u
            lax.fori_loop(0, NT // 4, quad, 0)
            for b in range(2):
                t = NT - 2 + b
                pltpu.make_async_copy(outb_r.at[b], acc_r.at[idxb_r.at[(t % 4), 1]],
                                      ss[b]).wait()
            plsc.subcore_barrier()
            pltpu.sync_copy(acc_r.at[pl.ds(s * RPT, RPT)],
                            out_hbm.at[pl.ds(s * RPT, RPT)])
            plsc.subcore_barrier()

        for core in range(2):
            @pl.when(c == core)
            def _():
                for k in range(cpc):
                    ch = core * cpc + k
                    run_chunk(hcs[ch], ecs[ch], outs[ch])

    fn = pl.kernel(
        body,
        out_type=tuple(jax.ShapeDtypeStruct((Np, LANE), jnp.float32)
                       for _ in range(nc)),
        mesh=mesh,
        scratch_types=[
            pltpu.VMEM((4, 2, T), jnp.int32),
            pltpu.VMEM((2, T, LANE), jnp.float32),
            pltpu.VMEM((2, T, LANE), jnp.float32),
            pltpu.VMEM((2, T, LANE), jnp.float32),
            pltpu.VMEM((ZR, LANE), jnp.float32),
            pltpu.VMEM_SHARED((Np, LANE), jnp.float32),
        ] + [pltpu.SemaphoreType.DMA] * 10,
    )
    return list(fn(idx2, *h_chunks, *ee_chunks))


def _edge_call(ea_p, W1, b1, W2, b2, elW0, elb0):
    Ep, DE = ea_p.shape
    EMH = W1.shape[1]
    d0 = elW0.shape[1]
    n0 = d0 // LANE
    grid = (Ep // EB,)

    def bodyfn(ea_r, W1_r, b1_r, W2_r, b2_r, eW_r, eb_r, e_r, *outs):
        t = jnp.maximum(
            jnp.dot(ea_r[...], W1_r[...], preferred_element_type=jnp.float32)
            + b1_r[...], 0.0)
        e = jnp.dot(t, W2_r[...], preferred_element_type=jnp.float32) + b2_r[...]
        e_r[...] = e
        ee = jnp.dot(e, eW_r[...], preferred_element_type=jnp.float32) + eb_r[...]
        for k, o in enumerate(outs):
            o[...] = ee[:, k * LANE:(k + 1) * LANE]

    return pl.pallas_call(
        bodyfn,
        grid=grid,
        in_specs=[
            pl.BlockSpec((EB, DE), lambda i: (i, 0)),
            pl.BlockSpec((DE, EMH), lambda i: (0, 0)),
            pl.BlockSpec((1, EMH), lambda i: (0, 0)),
            pl.BlockSpec((EMH, DE), lambda i: (0, 0)),
            pl.BlockSpec((1, DE), lambda i: (0, 0)),
            pl.BlockSpec((DE, d0), lambda i: (0, 0)),
            pl.BlockSpec((1, d0), lambda i: (0, 0)),
        ],
        out_specs=[pl.BlockSpec((EB, DE), lambda i: (i, 0))]
        + [pl.BlockSpec((EB, LANE), lambda i: (i, 0))] * n0,
        out_shape=[jax.ShapeDtypeStruct((Ep, DE), jnp.float32)]
        + [jax.ShapeDtypeStruct((Ep, LANE), jnp.float32)] * n0,
    )(ea_p, W1, b1, W2, b2, elW0, elb0)


def _ee_call(e_p, elW, elb):
    Ep, DE = e_p.shape
    d = elW.shape[1]
    n = d // LANE
    grid = (Ep // EB,)

    def bodyfn(e_r, W_r, b_r, *outs):
        ee = jnp.dot(e_r[...], W_r[...], preferred_element_type=jnp.float32) + b_r[...]
        for k, o in enumerate(outs):
            o[...] = ee[:, k * LANE:(k + 1) * LANE]

    return pl.pallas_call(
        bodyfn,
        grid=grid,
        in_specs=[
            pl.BlockSpec((EB, DE), lambda i: (i, 0)),
            pl.BlockSpec((DE, d), lambda i: (0, 0)),
            pl.BlockSpec((1, d), lambda i: (0, 0)),
        ],
        out_specs=[pl.BlockSpec((EB, LANE), lambda i: (i, 0))] * n,
        out_shape=[jax.ShapeDtypeStruct((Ep, LANE), jnp.float32)] * n,
    )(e_p, elW, elb)


def _node_call(h_chunks, a_chunks, W1, b1, W2, b2, W3p, b3p, eps2d, residual):
    nh = len(h_chunks)
    Np = h_chunks[0].shape[0]
    din = nh * LANE
    Hd = W1.shape[1]
    nout = Hd // LANE
    grid = (Np // NB,)

    def bodyfn(*refs):
        hrs = refs[:nh]
        ars = refs[nh:2 * nh]
        W1_r, b1_r, W2_r, b2_r, W3_r, b3_r, eps_r = refs[2 * nh:2 * nh + 7]
        outs = refs[2 * nh + 7:]
        h = jnp.concatenate([r[...] for r in hrs], axis=1)
        ag = jnp.concatenate([r[...] for r in ars], axis=1)
        z = (1.0 + eps_r[0, 0]) * h + ag
        z = jnp.maximum(
            jnp.dot(z, W1_r[...], preferred_element_type=jnp.float32) + b1_r[...], 0.0)
        z = jnp.maximum(
            jnp.dot(z, W2_r[...], preferred_element_type=jnp.float32) + b2_r[...], 0.0)
        z = jnp.dot(z, W3_r[...], preferred_element_type=jnp.float32) + b3_r[...]
        z = jnp.maximum(z, 0.0)
        if residual:
            z = h + z
        for k, o in enumerate(outs):
            o[...] = z[:, k * LANE:(k + 1) * LANE]

    return list(pl.pallas_call(
        bodyfn,
        grid=grid,
        in_specs=[pl.BlockSpec((NB, LANE), lambda i: (i, 0))] * (2 * nh)
        + [
            pl.BlockSpec((din, Hd), lambda i: (0, 0)),
            pl.BlockSpec((1, Hd), lambda i: (0, 0)),
            pl.BlockSpec((Hd, Hd), lambda i: (0, 0)),
            pl.BlockSpec((1, Hd), lambda i: (0, 0)),
            pl.BlockSpec((Hd, Hd), lambda i: (0, 0)),
            pl.BlockSpec((1, Hd), lambda i: (0, 0)),
            pl.BlockSpec((1, 1), lambda i: (0, 0)),
        ],
        out_specs=[pl.BlockSpec((NB, LANE), lambda i: (i, 0))] * nout,
        out_shape=[jax.ShapeDtypeStruct((Np, LANE), jnp.float32)] * nout,
    )(*h_chunks, *a_chunks, W1, b1, W2, b2, W3p, b3p, eps2d))


def _pool_call(h3_chunks, h2_chunks, batch2d, aW1, ab1, aW2, ab2, jW, jb):
    nh = len(h3_chunks)
    Np = h3_chunks[0].shape[0]
    Hd = nh * LANE
    Ah = aW1.shape[1]
    Jd = jW.shape[1]
    grid = (Np // NB,)

    def bodyfn(*refs):
        h3rs = refs[:nh]
        h2rs = refs[nh:2 * nh]
        b_r, aW1_r, ab1_r, aW2_r, ab2_r, jW_r, jb_r = refs[2 * nh:2 * nh + 7]
        pooled_r, jpool_r, counts_r, z_r = refs[2 * nh + 7:]
        i = pl.program_id(0)

        @pl.when(i == 0)
        def _():
            pooled_r[...] = jnp.zeros_like(pooled_r)
            jpool_r[...] = jnp.zeros_like(jpool_r)
            counts_r[...] = jnp.zeros_like(counts_r)
            z_r[...] = jnp.zeros_like(z_r)

        h3 = jnp.concatenate([r[...] for r in h3rs], axis=1)
        h2 = jnp.concatenate([r[...] for r in h2rs], axis=1)
        b = b_r[...]
        t = jnp.tanh(
            jnp.dot(h3, aW1_r[...], preferred_element_type=jnp.float32) + ab1_r[...])
        a = jnp.dot(t, aW2_r[...], preferred_element_type=jnp.float32) + ab2_r[...]
        ea = jnp.exp(a)
        valid = b < NUM_GRAPHS
        eam = jnp.where(valid, ea, 0.0)
        onehot = (b == lax.broadcasted_iota(jnp.int32, (NB, NUM_GRAPHS), 1)
                  ).astype(jnp.float32)
        hw = h3 * eam
        pooled_r[...] += lax.dot_general(
            onehot, hw, (((0,), (0,)), ((), ())),
            preferred_element_type=jnp.float32)
        jout = jnp.dot(h2, jW_r[...], preferred_element_type=jnp.float32) + jb_r[...]
        jpool_r[...] += lax.dot_general(
            onehot, jout, (((0,), (0,)), ((), ())),
            preferred_element_type=jnp.float32)
        counts_r[...] += jnp.sum(onehot, axis=0).reshape(NUM_GRAPHS, 1)
        z_r[...] += jnp.sum(eam).reshape(1, 1)

    cmap = lambda i: (0, 0)
    return pl.pallas_call(
        bodyfn,
        grid=grid,
        in_specs=[pl.BlockSpec((NB, LANE), lambda i: (i, 0))] * (2 * nh)
        + [
            pl.BlockSpec((NB, 1), lambda i: (i, 0)),
            pl.BlockSpec((Hd, Ah), cmap),
            pl.BlockSpec((1, Ah), cmap),
            pl.BlockSpec((Ah, 1), cmap),
            pl.BlockSpec((1, 1), cmap),
            pl.BlockSpec((Hd, Jd), cmap),
            pl.BlockSpec((1, Jd), cmap),
        ],
        out_specs=[
            pl.BlockSpec((NUM_GRAPHS, Hd), cmap),
            pl.BlockSpec((NUM_GRAPHS, Jd), cmap),
            pl.BlockSpec((NUM_GRAPHS, 1), cmap),
            pl.BlockSpec((1, 1), cmap),
        ],
        out_shape=[
            jax.ShapeDtypeStruct((NUM_GRAPHS, Hd), jnp.float32),
            jax.ShapeDtypeStruct((NUM_GRAPHS, Jd), jnp.float32),
            jax.ShapeDtypeStruct((NUM_GRAPHS, 1), jnp.float32),
            jax.ShapeDtypeStruct((1, 1), jnp.float32),
        ],
    )(*h3_chunks, *h2_chunks, batch2d, aW1, ab1, aW2, ab2, jW, jb)


def _head_call(pooled_u, jpool_u, counts, zsum, jcWa, jcWb, jcb,
               mW1, mb1, s1, t1, mW2, mb2, s2, t2, mW3, mb3, s3, t3,
               mW4, mb4, temp):
    OUTD = mW4.shape[1]

    def bodyfn(pool_r, jp_r, cnt_r, z_r, jcWa_r, jcWb_r, jcb_r,
               W1_r, b1_r, s1_r, t1_r, W2_r, b2_r, s2_r, t2_r,
               W3_r, b3_r, s3_r, t3_r, W4_r, b4_r, temp_r, out_r):
        pooled = pool_r[...] / z_r[0, 0]
        cnt = jnp.maximum(cnt_r[...], 1.0)
        jp = jp_r[...] / cnt
        xp = (jnp.dot(pooled, jcWa_r[...], preferred_element_type=jnp.float32)
              + jnp.dot(jp, jcWb_r[...], preferred_element_type=jnp.float32)
              + jcb_r[...])

        def bn_block(y, W_r, b_r, s_r, t_r):
            y = jnp.maximum(
                jnp.dot(y, W_r[...], preferred_element_type=jnp.float32) + b_r[...],
                0.0)
            return y * s_r[...] + t_r[...]

        y = bn_block(xp, W1_r, b1_r, s1_r, t1_r)
        y = bn_block(y, W2_r, b2_r, s2_r, t2_r)
        y = bn_block(y, W3_r, b3_r, s3_r, t3_r)
        logits = jnp.dot(y, W4_r[...], preferred_element_type=jnp.float32) + b4_r[...]
        out_r[...] = logits / temp_r[0, 0]

    return pl.pallas_call(
        bodyfn,
        out_shape=jax.ShapeDtypeStruct((NUM_GRAPHS, OUTD), jnp.float32),
    )(pooled_u, jpool_u, counts, zsum, jcWa, jcWb, jcb,
      mW1, mb1, s1, t1, mW2, mb2, s2, t2, mW3, mb3, s3, t3, mW4, mb4, temp)


def _chunks(arr):
    return [arr[:, k * LANE:(k + 1) * LANE] for k in range(arr.shape[1] // LANE)]


def _r2(v):
    return jnp.asarray(v, jnp.float32).reshape(1, -1)


def kernel(x, edge_index, edge_attr, batch, params):
    p = params
    N, D = x.shape
    E = edge_index.shape[1]
    Hd = p['g0_W1'].shape[1]

    Np = ((N + 2047) // 2048 + (1 if N % 2048 == 0 else 0)) * 2048
    epq = 16 * SC_TILE * 4
    while epq % EB:
        epq *= 2
    Ep = ((E + epq - 1) // epq) * epq

    src_p = jnp.concatenate(
        [edge_index[0], jnp.zeros((Ep - E,), jnp.int32)]).reshape(-1, SC_TILE)
    dst_p = jnp.concatenate(
        [edge_index[1], jnp.full((Ep - E,), N, jnp.int32)]).reshape(-1, SC_TILE)
    idx2 = jnp.stack([src_p, dst_p], axis=1)
    ea_p = jnp.concatenate(
        [edge_attr, jnp.zeros((Ep - E, edge_attr.shape[1]), jnp.float32)])
    x_p = jnp.concatenate([x, jnp.zeros((Np - N, D), jnp.float32)])
    batch_p = jnp.concatenate(
        [batch, jnp.full((Np - N,), NUM_GRAPHS, jnp.int32)]).reshape(Np, 1)

    e_p, *ee0c = _edge_call(ea_p, p['emlp_W1'], _r2(p['emlp_b1']),
                            p['emlp_W2'], _r2(p['emlp_b2']),
                            p['g0_el_W'], _r2(p['g0_el_b']))
    ee1c = list(_ee_call(e_p, p['g1_el_W'], _r2(p['g1_el_b'])))
    ee2c = list(_ee_call(e_p, p['g2_el_W'], _r2(p['g2_el_b'])))

    def fold_bn(i):
        scale = (p['g%d_bn_g' % i] / jnp.sqrt(1.0 + BN_EPS))
        W3p = p['g%d_W3' % i] * scale[None, :]
        b3p = p['g%d_b3' % i] * scale + p['g%d_bn_b' % i]
        return W3p, _r2(b3p)

    hc = _chunks(x_p)
    for i, eec in enumerate((ee0c, ee1c, ee2c)):
        aggr = _sc_aggregate(idx2, hc, eec)
        W3p, b3p = fold_bn(i)
        out = _node_call(hc, aggr,
                         p['g%d_W1' % i], _r2(p['g%d_b1' % i]),
                         p['g%d_W2' % i], _r2(p['g%d_b2' % i]),
                         W3p, b3p,
                         p['g%d_eps' % i].reshape(1, 1),
                         residual=(i > 0))
        if i == 1:
            h2c = out
        hc = out
    h3c = hc

    pooled_u, jpool_u, counts, zsum = _pool_call(
        h3c, h2c, batch_p,
        p['attn_W1'], _r2(p['attn_b1']), p['attn_W2'], _r2(p['attn_b2']),
        p['jump_W'], _r2(p['jump_b']))

    jc_Wa = p['jc_W'][:Hd]
    jc_Wb = p['jc_W'][Hd:]

    def mbn(i):
        s = (p['m_bn%d_g' % i] / jnp.sqrt(1.0 + BN_EPS))
        return _r2(s), _r2(p['m_bn%d_b' % i])

    s1, t1 = mbn(1)
    s2, t2 = mbn(2)
    s3, t3 = mbn(3)
    logits = _head_call(
        pooled_u, jpool_u, counts, zsum, jc_Wa, jc_Wb, _r2(p['jc_b']),
        p['m_W1'], _r2(p['m_b1']), s1, t1,
        p['m_W2'], _r2(p['m_b2']), s2, t2,
        p['m_W3'], _r2(p['m_b3']), s3, t3,
        p['m_W4'], _r2(p['m_b4']), p['temperature'].reshape(1, 1))
    return logits
